# baseline (speedup 1.0000x reference)
import functools

import jax
import jax.numpy as jnp
from jax import lax
from jax.experimental import pallas as pl
from jax.experimental.pallas import tpu as pltpu
from jax.experimental.pallas import tpu_sc as plsc

C = 128
N_NODES = 10000
K_NBRS = 16
B_ROWS = N_NODES * K_NBRS
PAD = 16
PADDED_ROWS = B_ROWS + 2 * PAD


_NW = 32
_PER_W = B_ROWS // _NW
_CB = 40
_NBUF = 5
_NCHUNK = _PER_W // _CB
_NSUPER = _NCHUNK // _NBUF
def _sc_gather_body(xT, yT, i1, i0, zpad, xi_out, yj_out,
                    idx1_v, idx0_v, xrows, yrows, zpad_v, gsem, osem):
  cid = lax.axis_index("c")
  sid = lax.axis_index("s")
  wid = sid * 2 + cid
  base = wid * _PER_W

  pltpu.sync_copy(i1.at[pl.ds(base, _PER_W)], idx1_v)
  pltpu.sync_copy(i0.at[pl.ds(base, _PER_W)], idx0_v)

  def drain_out_copies():
    for b in range(_NBUF):
      pltpu.make_async_copy(xi_out.at[pl.ds(0, _CB)], xrows.at[b], osem).wait()
      pltpu.make_async_copy(yj_out.at[pl.ds(0, _CB)], yrows.at[b], osem).wait()

  def super_step(s, carry):
    @pl.when(s > 0)
    def _():
      drain_out_copies()

    gh = []
    for b in range(_NBUF):
      off = (s * _NBUF + b) * _CB
      gh.append(pltpu.async_copy(
          xT.at[idx1_v.at[pl.ds(off, _CB)]], xrows.at[b], gsem))
      gh.append(pltpu.async_copy(
          yT.at[idx0_v.at[pl.ds(off, _CB)]], yrows.at[b], gsem))
    for h in gh:
      h.wait()
    for b in range(_NBUF):
      off = (s * _NBUF + b) * _CB
      dst = PAD + base + off
      pltpu.async_copy(xrows.at[b], xi_out.at[pl.ds(dst, _CB)], osem)
      pltpu.async_copy(yrows.at[b], yj_out.at[pl.ds(dst, _CB)], osem)
    return carry

  lax.fori_loop(0, _NSUPER, super_step, 0, unroll=False)
  drain_out_copies()

  @pl.when(wid == 0)
  def _zero_pads():
    pltpu.sync_copy(zpad, zpad_v)
    pltpu.sync_copy(zpad_v, xi_out.at[pl.ds(0, PAD)])
    pltpu.sync_copy(zpad_v, yj_out.at[pl.ds(0, PAD)])
    pltpu.sync_copy(zpad_v, xi_out.at[pl.ds(PAD + B_ROWS, PAD)])
    pltpu.sync_copy(zpad_v, yj_out.at[pl.ds(PAD + B_ROWS, PAD)])


def _sc_double_gather(xT, yT, i1, i0, zpad):
  mesh = plsc.VectorSubcoreMesh(core_axis_name="c", subcore_axis_name="s")
  out = jax.ShapeDtypeStruct((PADDED_ROWS, C), jnp.float32)
  kern = pl.kernel(
      _sc_gather_body,
      mesh=mesh,
      out_type=[out, out],
      scratch_types=[
          pltpu.VMEM((_PER_W,), jnp.int32),
          pltpu.VMEM((_PER_W,), jnp.int32),
          pltpu.VMEM((_NBUF, _CB, C), jnp.float32),
          pltpu.VMEM((_NBUF, _CB, C), jnp.float32),
          pltpu.VMEM((PAD, C), jnp.float32),
          pltpu.SemaphoreType.DMA,
          pltpu.SemaphoreType.DMA,
      ],
  )
  return kern(xT, yT, i1, i0, zpad)


TILE_N = 200
L = TILE_N * K_NBRS
N_TILES = N_NODES // TILE_N


def _conv_body(xiA, xiB, yjA, yjB, wc, maxv, minv, stats):
  t = pl.program_id(0)
  zrow = jnp.zeros((1, 2 * C), jnp.bfloat16)
  Pc = jnp.concatenate(
      [zrow,
       jnp.concatenate([xiA[...], yjA[...]], axis=1).astype(jnp.bfloat16),
       jnp.concatenate([xiB[...], yjB[...]], axis=1).astype(jnp.bfloat16),
       zrow], axis=0)

  p_iota = lax.broadcasted_iota(jnp.int32, (L + 34, 1), 0)
  mL = (p_iota % 16 != 0).astype(jnp.bfloat16)
  mR = (p_iota % 16 != 1).astype(jnp.bfloat16)
  PcL, PcR = Pc * mL, Pc * mR

  acc = jnp.zeros((L, C), jnp.float32)
  for di in (-1, 0, 1):
    for dj in (-1, 0, 1):
      st = 17 + 16 * di + dj
      tap = (di + 1) * 3 + (dj + 1)
      Sc = (PcL if dj == -1 else PcR if dj == 1 else Pc)[st:st + L]
      acc += jnp.dot(Sc, wc[tap], preferred_element_type=jnp.float32)

  m = acc.reshape(TILE_N, K_NBRS, C)
  maxv[...] = jnp.max(m, axis=1)
  minv[...] = jnp.min(m, axis=1)

  s0 = jnp.sum(acc, axis=0, keepdims=True)
  s1 = jnp.sum(acc * acc, axis=0, keepdims=True)
  srow = jnp.concatenate([s0, s1, jnp.zeros((6, C), jnp.float32)], axis=0)

  @pl.when(t == 0)
  def _init():
    stats[...] = srow

  @pl.when(t > 0)
  def _accum():
    stats[...] += srow


def _tc_conv(xi_g, yj_g, wc):
  grid = (N_TILES,)
  specA = pl.BlockSpec((L, C), lambda t: (t, 0))
  specB = pl.BlockSpec((32, C), lambda t: ((t + 1) * (L // 32), 0))
  specW = pl.BlockSpec((9, 2 * C, C), lambda t: (0, 0, 0))
  return pl.pallas_call(
      _conv_body,
      grid=grid,
      in_specs=[specA, specB, specA, specB, specW],
      out_specs=[
          pl.BlockSpec((TILE_N, C), lambda t: (t, 0)),
          pl.BlockSpec((TILE_N, C), lambda t: (t, 0)),
          pl.BlockSpec((8, C), lambda t: (0, 0)),
      ],
      out_shape=[
          jax.ShapeDtypeStruct((N_NODES, C), jnp.float32),
          jax.ShapeDtypeStruct((N_NODES, C), jnp.float32),
          jax.ShapeDtypeStruct((8, C), jnp.float32),
      ],
      compiler_params=pltpu.CompilerParams(
          dimension_semantics=("arbitrary",),
      ),
  )(xi_g, xiB_view(xi_g), yj_g, xiB_view(yj_g), wc)


def xiB_view(a):
  return a


_FT = 1000


def _finish_body(maxv, minv, stats, gamma, beta, cb, out):
  cnt = jnp.float32(B_ROWS)
  mean_c = stats[0:1, :] / cnt
  ex2 = stats[1:2, :] / cnt
  var = ex2 - mean_c * mean_c
  mean = mean_c + cb[...]
  a = gamma[...] / jnp.sqrt(var + 1e-5)
  bb = beta[...] - mean * a
  sel = jnp.where(a >= 0, maxv[...], minv[...])
  out[...] = jnp.maximum(sel * a + bb, 0.0)


def _tc_finish(maxv, minv, stats, gamma, beta, cb):
  grid = (N_NODES // _FT,)
  specT = pl.BlockSpec((_FT, C), lambda t: (t, 0))
  spec1 = pl.BlockSpec((1, C), lambda t: (0, 0))
  return pl.pallas_call(
      _finish_body,
      grid=grid,
      in_specs=[specT, specT, pl.BlockSpec((8, C), lambda t: (0, 0)),
                spec1, spec1, spec1],
      out_specs=specT,
      out_shape=jax.ShapeDtypeStruct((N_NODES, C), jnp.float32),
  )(maxv, minv, stats, gamma, beta, cb)


@jax.jit
def kernel(x, edge_index, y, conv_w, conv_b, bn_gamma, bn_beta):
  xT = jnp.transpose(x[0, :, :, 0])
  yT = jnp.transpose(y[0, :, :, 0])
  i1 = edge_index[1].reshape(-1).astype(jnp.int32)
  i0 = edge_index[0].reshape(-1).astype(jnp.int32)

  w1 = conv_w[:, :C]
  w2 = conv_w[:, C:]
  wx = jnp.transpose(w1 - w2, (2, 3, 1, 0)).reshape(9, C, C)
  wy = jnp.transpose(w2, (2, 3, 1, 0)).reshape(9, C, C)
  wc = jnp.concatenate([wx, wy], axis=1).astype(jnp.bfloat16)

  zpad = jnp.zeros((PAD, C), jnp.float32)
  xi_g, yj_g = _sc_double_gather(xT, yT, i1, i0, zpad)
  maxv, minv, stats = _tc_conv(xi_g, yj_g, wc)
  res = _tc_finish(maxv, minv, stats,
                   bn_gamma.reshape(1, C), bn_beta.reshape(1, C),
                   conv_b.reshape(1, C))
  return jnp.transpose(res)[None, :, :, None]


# speedup vs baseline: 1.0635x; 1.0635x over previous
import functools

import jax
import jax.numpy as jnp
from jax import lax
from jax.experimental import pallas as pl
from jax.experimental.pallas import tpu as pltpu
from jax.experimental.pallas import tpu_sc as plsc

C = 128
N_NODES = 10000
K_NBRS = 16
B_ROWS = N_NODES * K_NBRS
PAD = 16
H_ROWS = B_ROWS // 2
H_PADDED = H_ROWS + 2 * PAD


_NW = 20
_PER_W = H_ROWS // _NW
_CB = 40
_NBUF = 5
_NCHUNK = _PER_W // _CB
_NSUPER = _NCHUNK // _NBUF


def _make_sc_body(half):
  fbase = half * H_ROWS
  if half == 0:
    halo_src, halo_dst, zero_dst = H_ROWS, PAD + H_ROWS, 0
  else:
    halo_src, halo_dst, zero_dst = H_ROWS - PAD, 0, PAD + H_ROWS

  def body(xT, yT, i1, i0, zpad, xi_out, yj_out,
           idx1_v, idx0_v, xrows, yrows, hidx1_v, hidx0_v,
           hxrows, hyrows, zpad_v, gsem, osem):
    cid = lax.axis_index("c")
    sid = lax.axis_index("s")
    wid = sid * 2 + cid

    @pl.when(wid < _NW)
    def _main():
      base = wid * _PER_W

      pltpu.sync_copy(i1.at[pl.ds(fbase + base, _PER_W)], idx1_v)
      pltpu.sync_copy(i0.at[pl.ds(fbase + base, _PER_W)], idx0_v)

      def drain_out_copies():
        for b in range(_NBUF):
          pltpu.make_async_copy(
              xi_out.at[pl.ds(0, _CB)], xrows.at[b], osem).wait()
          pltpu.make_async_copy(
              yj_out.at[pl.ds(0, _CB)], yrows.at[b], osem).wait()

      def super_step(s, carry):
        @pl.when(s > 0)
        def _():
          drain_out_copies()
        gh = []
        for b in range(_NBUF):
          off = (s * _NBUF + b) * _CB
          gh.append(pltpu.async_copy(
              xT.at[idx1_v.at[pl.ds(off, _CB)]], xrows.at[b], gsem))
          gh.append(pltpu.async_copy(
              yT.at[idx0_v.at[pl.ds(off, _CB)]], yrows.at[b], gsem))
        for h in gh:
          h.wait()
        for b in range(_NBUF):
          off = (s * _NBUF + b) * _CB
          dst = PAD + base + off
          pltpu.async_copy(xrows.at[b], xi_out.at[pl.ds(dst, _CB)], osem)
          pltpu.async_copy(yrows.at[b], yj_out.at[pl.ds(dst, _CB)], osem)
        return carry

      lax.fori_loop(0, _NSUPER, super_step, 0, unroll=False)
      drain_out_copies()

    @pl.when(wid == _NW)
    def _halo():
      pltpu.sync_copy(i1.at[pl.ds(halo_src, PAD)], hidx1_v)
      pltpu.sync_copy(i0.at[pl.ds(halo_src, PAD)], hidx0_v)
      g1 = pltpu.async_copy(xT.at[hidx1_v], hxrows, gsem)
      g2 = pltpu.async_copy(yT.at[hidx0_v], hyrows, gsem)
      g1.wait()
      g2.wait()
      pltpu.sync_copy(hxrows, xi_out.at[pl.ds(halo_dst, PAD)])
      pltpu.sync_copy(hyrows, yj_out.at[pl.ds(halo_dst, PAD)])

    @pl.when(wid == _NW + 1)
    def _zero_pads():
      pltpu.sync_copy(zpad, zpad_v)
      pltpu.sync_copy(zpad_v, xi_out.at[pl.ds(zero_dst, PAD)])
      pltpu.sync_copy(zpad_v, yj_out.at[pl.ds(zero_dst, PAD)])

  return body


def _sc_double_gather(xT, yT, i1, i0, zpad, half):
  mesh = plsc.VectorSubcoreMesh(core_axis_name="c", subcore_axis_name="s")
  out = jax.ShapeDtypeStruct((H_PADDED, C), jnp.float32)
  kern = pl.kernel(
      _make_sc_body(half),
      mesh=mesh,
      out_type=[out, out],
      scratch_types=[
          pltpu.VMEM((_PER_W,), jnp.int32),
          pltpu.VMEM((_PER_W,), jnp.int32),
          pltpu.VMEM((_NBUF, _CB, C), jnp.float32),
          pltpu.VMEM((_NBUF, _CB, C), jnp.float32),
          pltpu.VMEM((PAD,), jnp.int32),
          pltpu.VMEM((PAD,), jnp.int32),
          pltpu.VMEM((PAD, C), jnp.float32),
          pltpu.VMEM((PAD, C), jnp.float32),
          pltpu.VMEM((PAD, C), jnp.float32),
          pltpu.SemaphoreType.DMA,
          pltpu.SemaphoreType.DMA,
      ],
  )
  return kern(xT, yT, i1, i0, zpad)


TILE_N = 200
L = TILE_N * K_NBRS
N_HALF = N_NODES // 2
N_TILES = N_HALF // TILE_N


def _conv_body(xiA, xiB, yjA, yjB, wc, maxv, minv, stats):
  t = pl.program_id(0)
  zrow = jnp.zeros((1, 2 * C), jnp.bfloat16)
  Pc = jnp.concatenate(
      [zrow,
       jnp.concatenate([xiA[...], yjA[...]], axis=1).astype(jnp.bfloat16),
       jnp.concatenate([xiB[...], yjB[...]], axis=1).astype(jnp.bfloat16),
       zrow], axis=0)

  p_iota = lax.broadcasted_iota(jnp.int32, (L + 34, 1), 0)
  mL = (p_iota % 16 != 0).astype(jnp.bfloat16)
  mR = (p_iota % 16 != 1).astype(jnp.bfloat16)
  QL = (Pc * mL)[0:L + 32]
  Q0 = Pc[1:L + 33]
  QR = (Pc * mR)[2:L + 34]

  acc = jnp.zeros((L, C), jnp.float32)
  for di in (-1, 0, 1):
    for dj in (-1, 0, 1):
      st = 16 + 16 * di
      tap = (di + 1) * 3 + (dj + 1)
      Sc = (QL if dj == -1 else QR if dj == 1 else Q0)[st:st + L]
      acc += jnp.dot(Sc, wc[tap], preferred_element_type=jnp.float32)

  m = acc.reshape(TILE_N, K_NBRS, C)
  maxv[...] = jnp.max(m, axis=1)
  minv[...] = jnp.min(m, axis=1)

  s0 = jnp.sum(acc, axis=0, keepdims=True)
  s1 = jnp.sum(acc * acc, axis=0, keepdims=True)
  srow = jnp.concatenate([s0, s1, jnp.zeros((6, C), jnp.float32)], axis=0)

  @pl.when(t == 0)
  def _init():
    stats[...] = srow

  @pl.when(t > 0)
  def _accum():
    stats[...] += srow


def _tc_conv(xi_g, yj_g, wc):
  grid = (N_TILES,)
  specA = pl.BlockSpec((L, C), lambda t: (t, 0))
  specB = pl.BlockSpec((32, C), lambda t: ((t + 1) * (L // 32), 0))
  specW = pl.BlockSpec((9, 2 * C, C), lambda t: (0, 0, 0))
  return pl.pallas_call(
      _conv_body,
      grid=grid,
      in_specs=[specA, specB, specA, specB, specW],
      out_specs=[
          pl.BlockSpec((TILE_N, C), lambda t: (t, 0)),
          pl.BlockSpec((TILE_N, C), lambda t: (t, 0)),
          pl.BlockSpec((8, C), lambda t: (0, 0)),
      ],
      out_shape=[
          jax.ShapeDtypeStruct((N_HALF, C), jnp.float32),
          jax.ShapeDtypeStruct((N_HALF, C), jnp.float32),
          jax.ShapeDtypeStruct((8, C), jnp.float32),
      ],
      compiler_params=pltpu.CompilerParams(
          dimension_semantics=("arbitrary",),
      ),
  )(xi_g, xi_g, yj_g, yj_g, wc)


_FT = 1000


def _finish_body(maxv, minv, statsA, statsB, gamma, beta, cb, out):
  cnt = jnp.float32(B_ROWS)
  stats = statsA[...] + statsB[...]
  mean_c = stats[0:1, :] / cnt
  ex2 = stats[1:2, :] / cnt
  var = ex2 - mean_c * mean_c
  mean = mean_c + cb[...]
  a = gamma[...] / jnp.sqrt(var + 1e-5)
  bb = beta[...] - mean * a
  sel = jnp.where(a >= 0, maxv[...], minv[...])
  out[...] = jnp.maximum(sel * a + bb, 0.0)


def _tc_finish(maxv, minv, statsA, statsB, gamma, beta, cb):
  grid = (N_HALF // _FT,)
  specT = pl.BlockSpec((_FT, C), lambda t: (t, 0))
  specS = pl.BlockSpec((8, C), lambda t: (0, 0))
  spec1 = pl.BlockSpec((1, C), lambda t: (0, 0))
  return pl.pallas_call(
      _finish_body,
      grid=grid,
      in_specs=[specT, specT, specS, specS, spec1, spec1, spec1],
      out_specs=specT,
      out_shape=jax.ShapeDtypeStruct((N_HALF, C), jnp.float32),
  )(maxv, minv, statsA, statsB, gamma, beta, cb)


@jax.jit
def kernel(x, edge_index, y, conv_w, conv_b, bn_gamma, bn_beta):
  xT = jnp.transpose(x[0, :, :, 0])
  yT = jnp.transpose(y[0, :, :, 0])
  i1 = edge_index[1].reshape(-1).astype(jnp.int32)
  i0 = edge_index[0].reshape(-1).astype(jnp.int32)

  w1 = conv_w[:, :C]
  w2 = conv_w[:, C:]
  wx = jnp.transpose(w1 - w2, (2, 3, 1, 0)).reshape(9, C, C)
  wy = jnp.transpose(w2, (2, 3, 1, 0)).reshape(9, C, C)
  wc = jnp.concatenate([wx, wy], axis=1).astype(jnp.bfloat16)

  zpad = jnp.zeros((PAD, C), jnp.float32)
  xi0, yj0 = _sc_double_gather(xT, yT, i1, i0, zpad, 0)
  xi1, yj1 = _sc_double_gather(xT, yT, i1, i0, zpad, 1)
  maxv0, minv0, stats0 = _tc_conv(xi0, yj0, wc)
  maxv1, minv1, stats1 = _tc_conv(xi1, yj1, wc)

  g = bn_gamma.reshape(1, C)
  b = bn_beta.reshape(1, C)
  cb = conv_b.reshape(1, C)
  res0 = _tc_finish(maxv0, minv0, stats0, stats1, g, b, cb)
  res1 = _tc_finish(maxv1, minv1, stats0, stats1, g, b, cb)
  res = jnp.concatenate([res0, res1], axis=0)
  return jnp.transpose(res)[None, :, :, None]
